# baseline (speedup 1.0000x reference)
import functools

import jax
import jax.numpy as jnp
from jax import lax
from jax.experimental import pallas as pl
from jax.experimental.pallas import tpu as pltpu
from jax.experimental.pallas import tpu_sc as plsc

N = 10000
E = 320000
D_IN = 128
D_H = 256
N_GPS = 128

NC = 2
NS = 16
NW = NC * NS
L = 16

G = 80
CPT = E // NW // G
CPS = E // NS // G
G2 = 125
CT2 = E // NW // G2
CS2 = E // NS // G2
B2 = 16
NPA = 10240
RPT = NPA // NS
NPD = 10240
EPT = E // NW


def _mesh():
    return plsc.VectorSubcoreMesh(core_axis_name="c", subcore_axis_name="s")


@functools.partial(
    pl.kernel,
    out_type=jax.ShapeDtypeStruct((NC, 1, NPD), jnp.float32),
    mesh=_mesh(),
    scratch_types=[
        pltpu.VMEM((CPT, G), jnp.int32),
        pltpu.VMEM((G,), jnp.float32),
        pltpu.VMEM((NPD // NS,), jnp.float32),
        pltpu.VMEM_SHARED((NPD,), jnp.float32),
    ],
)
def _deg_kernel(dst_hbm, out_hbm, idx_v, ones_v, zb_v, acc):
    c = lax.axis_index("c")
    s = lax.axis_index("s")
    wid = s * NC + c
    zn = NPD // NS

    @pl.loop(0, G // L)
    def _(i):
        ones_v[pl.ds(i * L, L)] = jnp.ones((L,), jnp.float32)

    @pl.loop(0, zn // L)
    def _(i):
        zb_v[pl.ds(i * L, L)] = jnp.zeros((L,), jnp.float32)

    pltpu.sync_copy(zb_v, acc.at[pl.ds(s * zn, zn)])
    pltpu.sync_copy(dst_hbm.at[wid], idx_v)
    plsc.subcore_barrier()

    @pl.loop(0, CPT)
    def _(g):
        pltpu.sync_copy(ones_v, acc.at[idx_v.at[g]], add=True)

    plsc.subcore_barrier()
    pltpu.sync_copy(acc.at[pl.ds(s * zn, zn)], out_hbm.at[c, 0, pl.ds(s * zn, zn)])


def _zero_acc(rows_v, acc, s):
    @pl.loop(0, 80)
    def _(r):
        for k in range(D_IN // L):
            rows_v[r, pl.ds(k * L, L)] = jnp.zeros((L,), jnp.float32)

    @pl.loop(0, RPT // 80)
    def _(j):
        pltpu.sync_copy(rows_v.at[pl.ds(0, 80)],
                        acc.at[pl.ds(s * RPT + j * 80, 80)])


def _agg_blocks(tbl, src_hbm, dst_hbm, row, nblk,
                src_v, dst_v, rows0, rows1, acc, sg0, sg1, ss0, ss1):
    @pl.loop(0, nblk)
    def _(blk):
        pltpu.sync_copy(src_hbm.at[row, pl.ds(blk * B2, B2)], src_v)
        pltpu.sync_copy(dst_hbm.at[row, pl.ds(blk * B2, B2)], dst_v)
        pltpu.async_copy(tbl.at[src_v.at[0]], rows0, sg0)

        @pl.loop(0, B2 // 2)
        def _(p):
            j0 = 2 * p
            j1 = 2 * p + 1

            @pl.when(p > 0)
            def _():
                pltpu.make_async_copy(rows1, acc.at[dst_v.at[j1 - 2]], ss1).wait()

            pltpu.async_copy(tbl.at[src_v.at[j1]], rows1, sg1)
            pltpu.make_async_copy(tbl.at[src_v.at[j0]], rows0, sg0).wait()
            pltpu.async_copy(rows0, acc.at[dst_v.at[j0]], ss0, add=True)
            pltpu.make_async_copy(rows0, acc.at[dst_v.at[j0]], ss0).wait()

            @pl.when(p < B2 // 2 - 1)
            def _():
                pltpu.async_copy(tbl.at[src_v.at[j0 + 2]], rows0, sg0)

            pltpu.make_async_copy(tbl.at[src_v.at[j1]], rows1, sg1).wait()
            pltpu.async_copy(rows1, acc.at[dst_v.at[j1]], ss1, add=True)

        pltpu.make_async_copy(rows1, acc.at[dst_v.at[B2 - 1]], ss1).wait()


@functools.partial(
    pl.kernel,
    out_type=jax.ShapeDtypeStruct((NC, NPA, D_IN), jnp.float32),
    mesh=_mesh(),
    scratch_types=[
        pltpu.VMEM((B2, G2), jnp.int32),
        pltpu.VMEM((B2, G2), jnp.int32),
        pltpu.VMEM((G2, D_IN), jnp.float32),
        pltpu.VMEM((G2, D_IN), jnp.float32),
        pltpu.VMEM_SHARED((NPA, D_IN), jnp.float32),
        pltpu.SemaphoreType.DMA,
        pltpu.SemaphoreType.DMA,
        pltpu.SemaphoreType.DMA,
        pltpu.SemaphoreType.DMA,
    ],
)
def _agg1_kernel(xs_hbm, src_hbm, dst_hbm, out_hbm,
                 src_v, dst_v, rows0, rows1, acc, sg0, sg1, ss0, ss1):
    c = lax.axis_index("c")
    s = lax.axis_index("s")
    wid = s * NC + c

    _zero_acc(rows0, acc, s)
    plsc.subcore_barrier()
    _agg_blocks(xs_hbm, src_hbm, dst_hbm, wid, CT2 // B2,
                src_v, dst_v, rows0, rows1, acc, sg0, sg1, ss0, ss1)
    plsc.subcore_barrier()
    pltpu.sync_copy(acc.at[pl.ds(s * RPT, RPT)], out_hbm.at[c, pl.ds(s * RPT, RPT)])


@functools.partial(
    pl.kernel,
    out_type=jax.ShapeDtypeStruct((NC, NPA, D_IN), jnp.float32),
    mesh=_mesh(),
    scratch_types=[
        pltpu.VMEM((B2, G2), jnp.int32),
        pltpu.VMEM((B2, G2), jnp.int32),
        pltpu.VMEM((G2, D_IN), jnp.float32),
        pltpu.VMEM((G2, D_IN), jnp.float32),
        pltpu.VMEM_SHARED((NPA, D_IN), jnp.float32),
        pltpu.SemaphoreType.DMA,
        pltpu.SemaphoreType.DMA,
        pltpu.SemaphoreType.DMA,
        pltpu.SemaphoreType.DMA,
    ],
)
def _agg2_kernel(hs_hbm, src_hbm, dst_hbm, out_hbm,
                 src_v, dst_v, rows0, rows1, acc, sg0, sg1, ss0, ss1):
    c = lax.axis_index("c")
    s = lax.axis_index("s")

    _zero_acc(rows0, acc, s)
    plsc.subcore_barrier()
    _agg_blocks(hs_hbm.at[c], src_hbm, dst_hbm, s, CS2 // B2,
                src_v, dst_v, rows0, rows1, acc, sg0, sg1, ss0, ss1)
    plsc.subcore_barrier()
    pltpu.sync_copy(acc.at[pl.ds(s * RPT, RPT)], out_hbm.at[c, pl.ds(s * RPT, RPT)])


PBR = G // 8
@functools.partial(
    pl.kernel,
    out_type=jax.ShapeDtypeStruct((NW, CPT, PBR, 128), jnp.float32),
    mesh=_mesh(),
    scratch_types=[
        pltpu.VMEM((CPT, G), jnp.int32),
        pltpu.VMEM((CPT, G), jnp.int32),
        pltpu.VMEM((G, N_GPS), jnp.float32),
        pltpu.VMEM((G, N_GPS), jnp.float32),
        pltpu.VMEM((G, N_GPS), jnp.float32),
        pltpu.VMEM((G, N_GPS), jnp.float32),
        pltpu.VMEM((PBR, 128), jnp.float32),
        pltpu.VMEM((PBR, 128), jnp.float32),
        pltpu.SemaphoreType.DMA,
        pltpu.SemaphoreType.DMA,
        pltpu.SemaphoreType.DMA,
        pltpu.SemaphoreType.DMA,
        pltpu.SemaphoreType.DMA,
        pltpu.SemaphoreType.DMA,
    ],
)
def _logits_kernel(mu_hbm, src_hbm, dst_hbm, out_hbm, src_v, dst_v,
                   ra0, rb0, ra1, rb1, pb0, pb1,
                   sa0, sb0, sa1, sb1, so0, so1):
    c = lax.axis_index("c")
    s = lax.axis_index("s")
    wid = s * NC + c

    pltpu.sync_copy(src_hbm.at[wid], src_v)
    pltpu.sync_copy(dst_hbm.at[wid], dst_v)

    def compute(ra, rb, pb):
        @pl.loop(0, PBR, unroll=2)
        def _(g8):
            for j in range(8):
                e = g8 * 8 + j
                acc = ra[e, pl.ds(0, L)] * rb[e, pl.ds(0, L)]
                for k in range(1, N_GPS // L):
                    acc = acc + ra[e, pl.ds(k * L, L)] * rb[e, pl.ds(k * L, L)]
                pb[g8, pl.ds(j * L, L)] = acc

    pltpu.async_copy(mu_hbm.at[src_v.at[0]], ra0, sa0)
    pltpu.async_copy(mu_hbm.at[dst_v.at[0]], rb0, sb0)

    @pl.loop(0, CPT // 2)
    def _(p):
        j0 = 2 * p
        j1 = 2 * p + 1

        pltpu.async_copy(mu_hbm.at[src_v.at[j1]], ra1, sa1)
        pltpu.async_copy(mu_hbm.at[dst_v.at[j1]], rb1, sb1)
        pltpu.make_async_copy(mu_hbm.at[src_v.at[j0]], ra0, sa0).wait()
        pltpu.make_async_copy(mu_hbm.at[dst_v.at[j0]], rb0, sb0).wait()

        @pl.when(p > 0)
        def _():
            pltpu.make_async_copy(pb0, out_hbm.at[wid, j0 - 2], so0).wait()

        compute(ra0, rb0, pb0)
        pltpu.async_copy(pb0, out_hbm.at[wid, j0], so0)
        pltpu.async_copy(mu_hbm.at[src_v.at[j0 + 2]], ra0, sa0)
        pltpu.async_copy(mu_hbm.at[dst_v.at[j0 + 2]], rb0, sb0)
        pltpu.make_async_copy(mu_hbm.at[src_v.at[j1]], ra1, sa1).wait()
        pltpu.make_async_copy(mu_hbm.at[dst_v.at[j1]], rb1, sb1).wait()

        @pl.when(p > 0)
        def _():
            pltpu.make_async_copy(pb1, out_hbm.at[wid, j1 - 2], so1).wait()

        compute(ra1, rb1, pb1)
        pltpu.async_copy(pb1, out_hbm.at[wid, j1], so1)

    pltpu.make_async_copy(mu_hbm.at[src_v.at[CPT - 1]], ra0, sa0).wait()
    pltpu.make_async_copy(mu_hbm.at[dst_v.at[CPT - 1]], rb0, sb0).wait()
    pltpu.make_async_copy(pb0, out_hbm.at[wid, CPT - 3], so0).wait()
    compute(ra0, rb0, pb0)
    pltpu.async_copy(pb0, out_hbm.at[wid, CPT - 1], so0)
    pltpu.make_async_copy(pb0, out_hbm.at[wid, CPT - 1], so0).wait()
    pltpu.make_async_copy(pb1, out_hbm.at[wid, CPT - 2], so1).wait()


_BR = 1000


def _prep_tc(x, deg_a, deg_b):
    def body(x_ref, da_ref, db_ref, xs_ref, dv_ref):
        dinv = lax.rsqrt(da_ref[...] + db_ref[...] + 1.0)
        dv_ref[...] = dinv
        xs_ref[...] = jnp.log1p(x_ref[...]) * dinv

    return pl.pallas_call(
        body,
        grid=(N // _BR,),
        in_specs=[
            pl.BlockSpec((_BR, D_IN), lambda i: (i, 0)),
            pl.BlockSpec((_BR, 1), lambda i: (i, 0)),
            pl.BlockSpec((_BR, 1), lambda i: (i, 0)),
        ],
        out_specs=[
            pl.BlockSpec((_BR, D_IN), lambda i: (i, 0)),
            pl.BlockSpec((_BR, 1), lambda i: (i, 0)),
        ],
        out_shape=[
            jax.ShapeDtypeStruct((N, D_IN), jnp.float32),
            jax.ShapeDtypeStruct((N, 1), jnp.float32),
        ],
    )(x, deg_a, deg_b)


def _hidden_tc(raw1, xs, dinv, W1, b1):
    def body(r_ref, xs_ref, dv_ref, w_ref, b_ref, hs_ref):
        dinv = dv_ref[...]
        agg1 = dinv * (r_ref[0] + r_ref[1] + xs_ref[...])
        h = jnp.dot(agg1, w_ref[...], preferred_element_type=jnp.float32)
        h = jnp.maximum(h + b_ref[...], 0.0)
        hs = h * dinv
        hs_ref[0] = hs[:, :D_IN]
        hs_ref[1] = hs[:, D_IN:]

    return pl.pallas_call(
        body,
        grid=(N // _BR,),
        in_specs=[
            pl.BlockSpec((NC, _BR, D_IN), lambda i: (0, i, 0)),
            pl.BlockSpec((_BR, D_IN), lambda i: (i, 0)),
            pl.BlockSpec((_BR, 1), lambda i: (i, 0)),
            pl.BlockSpec((D_IN, D_H), lambda i: (0, 0)),
            pl.BlockSpec((1, D_H), lambda i: (0, 0)),
        ],
        out_specs=[pl.BlockSpec((NC, _BR, D_IN), lambda i: (0, i, 0))],
        out_shape=[jax.ShapeDtypeStruct((NC, N, D_IN), jnp.float32)],
    )(raw1, xs, dinv, W1, b1)[0]


def _out_tc(raw2, hs2, dinv, Wmu, Wls, Wdec, mask):
    def body(r_ref, hs_ref, dv_ref, wmu_ref, wls_ref, wd_ref, m_ref,
             mu_ref, ls_ref, nb_ref):
        dinv = dv_ref[...]
        a_lo = dinv * (r_ref[0] + hs_ref[0])
        a_hi = dinv * (r_ref[1] + hs_ref[1])
        wmu = wmu_ref[...]
        wls = wls_ref[...]
        mu = jnp.dot(a_lo, wmu[:D_IN], preferred_element_type=jnp.float32)
        mu = mu + jnp.dot(a_hi, wmu[D_IN:], preferred_element_type=jnp.float32)
        ls = jnp.dot(a_lo, wls[:D_IN], preferred_element_type=jnp.float32)
        ls = ls + jnp.dot(a_hi, wls[D_IN:], preferred_element_type=jnp.float32)
        mu_ref[...] = mu
        ls_ref[...] = ls
        wm = wd_ref[...] * m_ref[...]
        nb = jnp.dot(mu, wm, preferred_element_type=jnp.float32)
        nb_ref[...] = jnp.exp(jnp.clip(nb, -10.0, 10.0))

    return pl.pallas_call(
        body,
        grid=(N // _BR,),
        in_specs=[
            pl.BlockSpec((NC, _BR, D_IN), lambda i: (0, i, 0)),
            pl.BlockSpec((NC, _BR, D_IN), lambda i: (0, i, 0)),
            pl.BlockSpec((_BR, 1), lambda i: (i, 0)),
            pl.BlockSpec((D_H, N_GPS), lambda i: (0, 0)),
            pl.BlockSpec((D_H, N_GPS), lambda i: (0, 0)),
            pl.BlockSpec((N_GPS, D_IN), lambda i: (0, 0)),
            pl.BlockSpec((N_GPS, D_IN), lambda i: (0, 0)),
        ],
        out_specs=[
            pl.BlockSpec((_BR, N_GPS), lambda i: (i, 0)),
            pl.BlockSpec((_BR, N_GPS), lambda i: (i, 0)),
            pl.BlockSpec((_BR, D_IN), lambda i: (i, 0)),
        ],
        out_shape=[
            jax.ShapeDtypeStruct((N, N_GPS), jnp.float32),
            jax.ShapeDtypeStruct((N, N_GPS), jnp.float32),
            jax.ShapeDtypeStruct((N, D_IN), jnp.float32),
        ],
    )(raw2, hs2, dinv, Wmu, Wls, Wdec, mask)


def _lsum_tc(partial):
    _BE = 4000

    def body(p_ref, o_ref):
        d = lax.broadcasted_iota(jnp.int32, (128, 8), 0)
        j = lax.broadcasted_iota(jnp.int32, (128, 8), 1)
        sel = (d // L == j).astype(jnp.float32)
        o_ref[...] = jnp.dot(p_ref[...], sel, preferred_element_type=jnp.float32)

    return pl.pallas_call(
        body,
        grid=(E // 8 // _BE,),
        in_specs=[pl.BlockSpec((_BE, 128), lambda i: (i, 0))],
        out_specs=[pl.BlockSpec((_BE, 8), lambda i: (i, 0))],
        out_shape=[jax.ShapeDtypeStruct((E // 8, 8), jnp.float32)],
    )(partial)[0]


def kernel(x, edge_index, W1, b1, Wmu, Wls, Wdec, mask):
    src = edge_index[0].astype(jnp.int32)
    dst = edge_index[1].astype(jnp.int32)
    src32 = src.reshape(NW, CPT, G)
    dst32 = dst.reshape(NW, CPT, G)
    src32a = src.reshape(NW, CT2, G2)
    dst32a = dst.reshape(NW, CT2, G2)
    src16a = src.reshape(NS, CS2, G2)
    dst16a = dst.reshape(NS, CS2, G2)
    b1r = b1.reshape(1, D_H)

    deg2 = _deg_kernel(dst32)
    deg_a = deg2[0, 0, :N].reshape(N, 1)
    deg_b = deg2[1, 0, :N].reshape(N, 1)
    xs, dinv = _prep_tc(x, deg_a, deg_b)
    raw1 = _agg1_kernel(xs, src32a, dst32a)
    hs2 = _hidden_tc(raw1, xs, dinv, W1, b1r)
    raw2 = _agg2_kernel(hs2, src16a, dst16a)
    mu, logstd, nb_mean = _out_tc(raw2, hs2, dinv, Wmu, Wls, Wdec, mask)
    lpart = _logits_kernel(mu, src32, dst32).reshape(E // 8, 128)
    edge_logits = _lsum_tc(lpart).reshape(E)
    return mu, logstd, edge_logits, nb_mean


# speedup vs baseline: 1.0137x; 1.0137x over previous
import functools

import jax
import jax.numpy as jnp
from jax import lax
from jax.experimental import pallas as pl
from jax.experimental.pallas import tpu as pltpu
from jax.experimental.pallas import tpu_sc as plsc

N = 10000
E = 320000
D_IN = 128
D_H = 256
N_GPS = 128

NC = 2
NS = 16
NW = NC * NS
L = 16

G = 80
CPT = E // NW // G
CPS = E // NS // G
G2 = 125
CT2 = E // NW // G2
CS2 = E // NS // G2
B2 = 16
NPA = 10240
RPT = NPA // NS
NPD = 10240
EPT = E // NW


def _mesh():
    return plsc.VectorSubcoreMesh(core_axis_name="c", subcore_axis_name="s")


@functools.partial(
    pl.kernel,
    out_type=jax.ShapeDtypeStruct((NC, 1, NPD), jnp.float32),
    mesh=_mesh(),
    scratch_types=[
        pltpu.VMEM((CPT, G), jnp.int32),
        pltpu.VMEM((G,), jnp.float32),
        pltpu.VMEM((NPD // NS,), jnp.float32),
        pltpu.VMEM_SHARED((NPD,), jnp.float32),
    ],
)
def _deg_kernel(dst_hbm, out_hbm, idx_v, ones_v, zb_v, acc):
    c = lax.axis_index("c")
    s = lax.axis_index("s")
    wid = s * NC + c
    zn = NPD // NS

    @pl.loop(0, G // L)
    def _(i):
        ones_v[pl.ds(i * L, L)] = jnp.ones((L,), jnp.float32)

    @pl.loop(0, zn // L)
    def _(i):
        zb_v[pl.ds(i * L, L)] = jnp.zeros((L,), jnp.float32)

    pltpu.sync_copy(zb_v, acc.at[pl.ds(s * zn, zn)])
    pltpu.sync_copy(dst_hbm.at[wid], idx_v)
    plsc.subcore_barrier()

    @pl.loop(0, CPT)
    def _(g):
        pltpu.sync_copy(ones_v, acc.at[idx_v.at[g]], add=True)

    plsc.subcore_barrier()
    pltpu.sync_copy(acc.at[pl.ds(s * zn, zn)], out_hbm.at[c, 0, pl.ds(s * zn, zn)])


def _zero_acc(rows_v, acc, s):
    @pl.loop(0, 80)
    def _(r):
        for k in range(D_IN // L):
            rows_v[r, pl.ds(k * L, L)] = jnp.zeros((L,), jnp.float32)

    @pl.loop(0, RPT // 80)
    def _(j):
        pltpu.sync_copy(rows_v.at[pl.ds(0, 80)],
                        acc.at[pl.ds(s * RPT + j * 80, 80)])


BQ = 8


def _agg_blocks(tbl, src_hbm, dst_hbm, row, nblk,
                sA, dA, sB, dB, rows0, rows1, acc, sg0, sg1, ss0, ss1):
    pltpu.sync_copy(src_hbm.at[row, pl.ds(0, BQ)], sA)
    pltpu.sync_copy(dst_hbm.at[row, pl.ds(0, BQ)], dA)
    pltpu.async_copy(tbl.at[sA.at[0]], rows0, sg0)

    @pl.loop(0, nblk // 2)
    def _(sb):
        for bb in range(2):
            b = 2 * sb + bb
            sI, dI = (sA, dA) if bb == 0 else (sB, dB)
            sO, dO = (sB, dB) if bb == 0 else (sA, dA)
            for p in range(BQ // 2):
                j0 = 2 * p
                j1 = 2 * p + 1

                if p == 0:
                    @pl.when(b > 0)
                    def _():
                        pltpu.make_async_copy(
                            rows1, acc.at[dO.at[BQ - 1]], ss1).wait()

                    @pl.when(b + 1 < nblk)
                    def _():
                        pltpu.sync_copy(
                            src_hbm.at[row, pl.ds((b + 1) * BQ, BQ)], sO)
                        pltpu.sync_copy(
                            dst_hbm.at[row, pl.ds((b + 1) * BQ, BQ)], dO)
                else:
                    pltpu.make_async_copy(
                        rows1, acc.at[dI.at[j1 - 2]], ss1).wait()

                pltpu.async_copy(tbl.at[sI.at[j1]], rows1, sg1)
                pltpu.make_async_copy(tbl.at[sI.at[j0]], rows0, sg0).wait()
                pltpu.async_copy(rows0, acc.at[dI.at[j0]], ss0, add=True)
                pltpu.make_async_copy(rows0, acc.at[dI.at[j0]], ss0).wait()

                if p < BQ // 2 - 1:
                    pltpu.async_copy(tbl.at[sI.at[j0 + 2]], rows0, sg0)
                else:
                    @pl.when(b + 1 < nblk)
                    def _():
                        pltpu.async_copy(tbl.at[sO.at[0]], rows0, sg0)

                pltpu.make_async_copy(tbl.at[sI.at[j1]], rows1, sg1).wait()
                pltpu.async_copy(rows1, acc.at[dI.at[j1]], ss1, add=True)

    pltpu.make_async_copy(rows1, acc.at[dB.at[BQ - 1]], ss1).wait()


@functools.partial(
    pl.kernel,
    out_type=jax.ShapeDtypeStruct((NC, NPA, D_IN), jnp.float32),
    mesh=_mesh(),
    scratch_types=[
        pltpu.VMEM((BQ, G2), jnp.int32),
        pltpu.VMEM((BQ, G2), jnp.int32),
        pltpu.VMEM((BQ, G2), jnp.int32),
        pltpu.VMEM((BQ, G2), jnp.int32),
        pltpu.VMEM((G2, D_IN), jnp.float32),
        pltpu.VMEM((G2, D_IN), jnp.float32),
        pltpu.VMEM_SHARED((NPA, D_IN), jnp.float32),
        pltpu.SemaphoreType.DMA,
        pltpu.SemaphoreType.DMA,
        pltpu.SemaphoreType.DMA,
        pltpu.SemaphoreType.DMA,
    ],
)
def _agg1_kernel(xs_hbm, src_hbm, dst_hbm, out_hbm,
                 sA, dA, sB, dB, rows0, rows1, acc, sg0, sg1, ss0, ss1):
    c = lax.axis_index("c")
    s = lax.axis_index("s")
    wid = s * NC + c

    _zero_acc(rows0, acc, s)
    plsc.subcore_barrier()
    _agg_blocks(xs_hbm, src_hbm, dst_hbm, wid, CT2 // BQ,
                sA, dA, sB, dB, rows0, rows1, acc, sg0, sg1, ss0, ss1)
    plsc.subcore_barrier()
    pltpu.sync_copy(acc.at[pl.ds(s * RPT, RPT)], out_hbm.at[c, pl.ds(s * RPT, RPT)])


@functools.partial(
    pl.kernel,
    out_type=jax.ShapeDtypeStruct((NC, NPA, D_IN), jnp.float32),
    mesh=_mesh(),
    scratch_types=[
        pltpu.VMEM((BQ, G2), jnp.int32),
        pltpu.VMEM((BQ, G2), jnp.int32),
        pltpu.VMEM((BQ, G2), jnp.int32),
        pltpu.VMEM((BQ, G2), jnp.int32),
        pltpu.VMEM((G2, D_IN), jnp.float32),
        pltpu.VMEM((G2, D_IN), jnp.float32),
        pltpu.VMEM_SHARED((NPA, D_IN), jnp.float32),
        pltpu.SemaphoreType.DMA,
        pltpu.SemaphoreType.DMA,
        pltpu.SemaphoreType.DMA,
        pltpu.SemaphoreType.DMA,
    ],
)
def _agg2_kernel(hs_hbm, src_hbm, dst_hbm, out_hbm,
                 sA, dA, sB, dB, rows0, rows1, acc, sg0, sg1, ss0, ss1):
    c = lax.axis_index("c")
    s = lax.axis_index("s")

    _zero_acc(rows0, acc, s)
    plsc.subcore_barrier()
    _agg_blocks(hs_hbm.at[c], src_hbm, dst_hbm, s, CS2 // BQ,
                sA, dA, sB, dB, rows0, rows1, acc, sg0, sg1, ss0, ss1)
    plsc.subcore_barrier()
    pltpu.sync_copy(acc.at[pl.ds(s * RPT, RPT)], out_hbm.at[c, pl.ds(s * RPT, RPT)])


PBR = G // 8
@functools.partial(
    pl.kernel,
    out_type=jax.ShapeDtypeStruct((NW, CPT, PBR, 128), jnp.float32),
    mesh=_mesh(),
    scratch_types=[
        pltpu.VMEM((CPT, G), jnp.int32),
        pltpu.VMEM((CPT, G), jnp.int32),
        pltpu.VMEM((G, N_GPS), jnp.float32),
        pltpu.VMEM((G, N_GPS), jnp.float32),
        pltpu.VMEM((G, N_GPS), jnp.float32),
        pltpu.VMEM((G, N_GPS), jnp.float32),
        pltpu.VMEM((PBR, 128), jnp.float32),
        pltpu.VMEM((PBR, 128), jnp.float32),
        pltpu.SemaphoreType.DMA,
        pltpu.SemaphoreType.DMA,
        pltpu.SemaphoreType.DMA,
        pltpu.SemaphoreType.DMA,
        pltpu.SemaphoreType.DMA,
        pltpu.SemaphoreType.DMA,
    ],
)
def _logits_kernel(mu_hbm, src_hbm, dst_hbm, out_hbm, src_v, dst_v,
                   ra0, rb0, ra1, rb1, pb0, pb1,
                   sa0, sb0, sa1, sb1, so0, so1):
    c = lax.axis_index("c")
    s = lax.axis_index("s")
    wid = s * NC + c

    pltpu.sync_copy(src_hbm.at[wid], src_v)
    pltpu.sync_copy(dst_hbm.at[wid], dst_v)

    def compute(ra, rb, pb):
        @pl.loop(0, PBR, unroll=2)
        def _(g8):
            for j in range(8):
                e = g8 * 8 + j
                acc = ra[e, pl.ds(0, L)] * rb[e, pl.ds(0, L)]
                for k in range(1, N_GPS // L):
                    acc = acc + ra[e, pl.ds(k * L, L)] * rb[e, pl.ds(k * L, L)]
                pb[g8, pl.ds(j * L, L)] = acc

    pltpu.async_copy(mu_hbm.at[src_v.at[0]], ra0, sa0)
    pltpu.async_copy(mu_hbm.at[dst_v.at[0]], rb0, sb0)

    @pl.loop(0, CPT // 2)
    def _(p):
        j0 = 2 * p
        j1 = 2 * p + 1

        pltpu.async_copy(mu_hbm.at[src_v.at[j1]], ra1, sa1)
        pltpu.async_copy(mu_hbm.at[dst_v.at[j1]], rb1, sb1)
        pltpu.make_async_copy(mu_hbm.at[src_v.at[j0]], ra0, sa0).wait()
        pltpu.make_async_copy(mu_hbm.at[dst_v.at[j0]], rb0, sb0).wait()

        @pl.when(p > 0)
        def _():
            pltpu.make_async_copy(pb0, out_hbm.at[wid, j0 - 2], so0).wait()

        compute(ra0, rb0, pb0)
        pltpu.async_copy(pb0, out_hbm.at[wid, j0], so0)
        pltpu.async_copy(mu_hbm.at[src_v.at[j0 + 2]], ra0, sa0)
        pltpu.async_copy(mu_hbm.at[dst_v.at[j0 + 2]], rb0, sb0)
        pltpu.make_async_copy(mu_hbm.at[src_v.at[j1]], ra1, sa1).wait()
        pltpu.make_async_copy(mu_hbm.at[dst_v.at[j1]], rb1, sb1).wait()

        @pl.when(p > 0)
        def _():
            pltpu.make_async_copy(pb1, out_hbm.at[wid, j1 - 2], so1).wait()

        compute(ra1, rb1, pb1)
        pltpu.async_copy(pb1, out_hbm.at[wid, j1], so1)

    pltpu.make_async_copy(mu_hbm.at[src_v.at[CPT - 1]], ra0, sa0).wait()
    pltpu.make_async_copy(mu_hbm.at[dst_v.at[CPT - 1]], rb0, sb0).wait()
    pltpu.make_async_copy(pb0, out_hbm.at[wid, CPT - 3], so0).wait()
    compute(ra0, rb0, pb0)
    pltpu.async_copy(pb0, out_hbm.at[wid, CPT - 1], so0)
    pltpu.make_async_copy(pb0, out_hbm.at[wid, CPT - 1], so0).wait()
    pltpu.make_async_copy(pb1, out_hbm.at[wid, CPT - 2], so1).wait()


_BR = 1000


def _prep_tc(x, deg_a, deg_b):
    def body(x_ref, da_ref, db_ref, xs_ref, dv_ref):
        dinv = lax.rsqrt(da_ref[...] + db_ref[...] + 1.0)
        dv_ref[...] = dinv
        xs_ref[...] = jnp.log1p(x_ref[...]) * dinv

    return pl.pallas_call(
        body,
        grid=(N // _BR,),
        in_specs=[
            pl.BlockSpec((_BR, D_IN), lambda i: (i, 0)),
            pl.BlockSpec((_BR, 1), lambda i: (i, 0)),
            pl.BlockSpec((_BR, 1), lambda i: (i, 0)),
        ],
        out_specs=[
            pl.BlockSpec((_BR, D_IN), lambda i: (i, 0)),
            pl.BlockSpec((_BR, 1), lambda i: (i, 0)),
        ],
        out_shape=[
            jax.ShapeDtypeStruct((N, D_IN), jnp.float32),
            jax.ShapeDtypeStruct((N, 1), jnp.float32),
        ],
    )(x, deg_a, deg_b)


def _hidden_tc(raw1, xs, dinv, W1, b1):
    def body(r_ref, xs_ref, dv_ref, w_ref, b_ref, hs_ref):
        dinv = dv_ref[...]
        agg1 = dinv * (r_ref[0] + r_ref[1] + xs_ref[...])
        h = jnp.dot(agg1, w_ref[...], preferred_element_type=jnp.float32)
        h = jnp.maximum(h + b_ref[...], 0.0)
        hs = h * dinv
        hs_ref[0] = hs[:, :D_IN]
        hs_ref[1] = hs[:, D_IN:]

    return pl.pallas_call(
        body,
        grid=(N // _BR,),
        in_specs=[
            pl.BlockSpec((NC, _BR, D_IN), lambda i: (0, i, 0)),
            pl.BlockSpec((_BR, D_IN), lambda i: (i, 0)),
            pl.BlockSpec((_BR, 1), lambda i: (i, 0)),
            pl.BlockSpec((D_IN, D_H), lambda i: (0, 0)),
            pl.BlockSpec((1, D_H), lambda i: (0, 0)),
        ],
        out_specs=[pl.BlockSpec((NC, _BR, D_IN), lambda i: (0, i, 0))],
        out_shape=[jax.ShapeDtypeStruct((NC, N, D_IN), jnp.float32)],
    )(raw1, xs, dinv, W1, b1)[0]


def _out_tc(raw2, hs2, dinv, Wmu, Wls, Wdec, mask):
    def body(r_ref, hs_ref, dv_ref, wmu_ref, wls_ref, wd_ref, m_ref,
             mu_ref, ls_ref, nb_ref):
        dinv = dv_ref[...]
        a_lo = dinv * (r_ref[0] + hs_ref[0])
        a_hi = dinv * (r_ref[1] + hs_ref[1])
        wmu = wmu_ref[...]
        wls = wls_ref[...]
        mu = jnp.dot(a_lo, wmu[:D_IN], preferred_element_type=jnp.float32)
        mu = mu + jnp.dot(a_hi, wmu[D_IN:], preferred_element_type=jnp.float32)
        ls = jnp.dot(a_lo, wls[:D_IN], preferred_element_type=jnp.float32)
        ls = ls + jnp.dot(a_hi, wls[D_IN:], preferred_element_type=jnp.float32)
        mu_ref[...] = mu
        ls_ref[...] = ls
        wm = wd_ref[...] * m_ref[...]
        nb = jnp.dot(mu, wm, preferred_element_type=jnp.float32)
        nb_ref[...] = jnp.exp(jnp.clip(nb, -10.0, 10.0))

    return pl.pallas_call(
        body,
        grid=(N // _BR,),
        in_specs=[
            pl.BlockSpec((NC, _BR, D_IN), lambda i: (0, i, 0)),
            pl.BlockSpec((NC, _BR, D_IN), lambda i: (0, i, 0)),
            pl.BlockSpec((_BR, 1), lambda i: (i, 0)),
            pl.BlockSpec((D_H, N_GPS), lambda i: (0, 0)),
            pl.BlockSpec((D_H, N_GPS), lambda i: (0, 0)),
            pl.BlockSpec((N_GPS, D_IN), lambda i: (0, 0)),
            pl.BlockSpec((N_GPS, D_IN), lambda i: (0, 0)),
        ],
        out_specs=[
            pl.BlockSpec((_BR, N_GPS), lambda i: (i, 0)),
            pl.BlockSpec((_BR, N_GPS), lambda i: (i, 0)),
            pl.BlockSpec((_BR, D_IN), lambda i: (i, 0)),
        ],
        out_shape=[
            jax.ShapeDtypeStruct((N, N_GPS), jnp.float32),
            jax.ShapeDtypeStruct((N, N_GPS), jnp.float32),
            jax.ShapeDtypeStruct((N, D_IN), jnp.float32),
        ],
    )(raw2, hs2, dinv, Wmu, Wls, Wdec, mask)


def _lsum_tc(partial):
    _BE = 4000

    def body(p_ref, o_ref):
        d = lax.broadcasted_iota(jnp.int32, (128, 8), 0)
        j = lax.broadcasted_iota(jnp.int32, (128, 8), 1)
        sel = (d // L == j).astype(jnp.float32)
        o_ref[...] = jnp.dot(p_ref[...], sel, preferred_element_type=jnp.float32)

    return pl.pallas_call(
        body,
        grid=(E // 8 // _BE,),
        in_specs=[pl.BlockSpec((_BE, 128), lambda i: (i, 0))],
        out_specs=[pl.BlockSpec((_BE, 8), lambda i: (i, 0))],
        out_shape=[jax.ShapeDtypeStruct((E // 8, 8), jnp.float32)],
    )(partial)[0]


def kernel(x, edge_index, W1, b1, Wmu, Wls, Wdec, mask):
    src = edge_index[0].astype(jnp.int32)
    dst = edge_index[1].astype(jnp.int32)
    src32 = src.reshape(NW, CPT, G)
    dst32 = dst.reshape(NW, CPT, G)
    src32a = src.reshape(NW, CT2, G2)
    dst32a = dst.reshape(NW, CT2, G2)
    src16a = src.reshape(NS, CS2, G2)
    dst16a = dst.reshape(NS, CS2, G2)
    b1r = b1.reshape(1, D_H)

    deg2 = _deg_kernel(dst32)
    deg_a = deg2[0, 0, :N].reshape(N, 1)
    deg_b = deg2[1, 0, :N].reshape(N, 1)
    xs, dinv = _prep_tc(x, deg_a, deg_b)
    raw1 = _agg1_kernel(xs, src32a, dst32a)
    hs2 = _hidden_tc(raw1, xs, dinv, W1, b1r)
    raw2 = _agg2_kernel(hs2, src16a, dst16a)
    mu, logstd, nb_mean = _out_tc(raw2, hs2, dinv, Wmu, Wls, Wdec, mask)
    lpart = _logits_kernel(mu, src32, dst32).reshape(E // 8, 128)
    edge_logits = _lsum_tc(lpart).reshape(E)
    return mu, logstd, edge_logits, nb_mean


# speedup vs baseline: 1.0420x; 1.0280x over previous
import functools

import jax
import jax.numpy as jnp
from jax import lax
from jax.experimental import pallas as pl
from jax.experimental.pallas import tpu as pltpu
from jax.experimental.pallas import tpu_sc as plsc

N = 10000
E = 320000
D_IN = 128
D_H = 256
N_GPS = 128

NC = 2
NS = 16
NW = NC * NS
L = 16

G = 80
CPT = E // NW // G
CPS = E // NS // G
G2 = 125
CT2 = E // NW // G2
CS2 = E // NS // G2
B2 = 16
NPA = 10240
RPT = NPA // NS
NPD = 10240
EPT = E // NW


def _mesh():
    return plsc.VectorSubcoreMesh(core_axis_name="c", subcore_axis_name="s")


@functools.partial(
    pl.kernel,
    out_type=jax.ShapeDtypeStruct((NC, 1, NPD), jnp.float32),
    mesh=_mesh(),
    scratch_types=[
        pltpu.VMEM((CPT, G), jnp.int32),
        pltpu.VMEM((G,), jnp.float32),
        pltpu.VMEM((NPD // NS,), jnp.float32),
        pltpu.VMEM_SHARED((NPD,), jnp.float32),
    ],
)
def _deg_kernel(dst_hbm, out_hbm, idx_v, ones_v, zb_v, acc):
    c = lax.axis_index("c")
    s = lax.axis_index("s")
    wid = s * NC + c
    zn = NPD // NS

    @pl.loop(0, G // L)
    def _(i):
        ones_v[pl.ds(i * L, L)] = jnp.ones((L,), jnp.float32)

    @pl.loop(0, zn // L)
    def _(i):
        zb_v[pl.ds(i * L, L)] = jnp.zeros((L,), jnp.float32)

    pltpu.sync_copy(zb_v, acc.at[pl.ds(s * zn, zn)])
    pltpu.sync_copy(dst_hbm.at[wid], idx_v)
    plsc.subcore_barrier()

    @pl.loop(0, CPT)
    def _(g):
        pltpu.sync_copy(ones_v, acc.at[idx_v.at[g]], add=True)

    plsc.subcore_barrier()
    pltpu.sync_copy(acc.at[pl.ds(s * zn, zn)], out_hbm.at[c, 0, pl.ds(s * zn, zn)])


def _zero_acc(rows_v, acc, s):
    @pl.loop(0, 80)
    def _(r):
        for k in range(D_IN // L):
            rows_v[r, pl.ds(k * L, L)] = jnp.zeros((L,), jnp.float32)

    @pl.loop(0, RPT // 80)
    def _(j):
        pltpu.sync_copy(rows_v.at[pl.ds(0, 80)],
                        acc.at[pl.ds(s * RPT + j * 80, 80)])


BQ = 8


def _agg_blocks(tbl, src_hbm, dst_hbm, row, nblk,
                sA, dA, sB, dB, rows0, rows1, acc, sg0, sg1, ss0, ss1):
    pltpu.sync_copy(src_hbm.at[row, pl.ds(0, BQ)], sA)
    pltpu.sync_copy(dst_hbm.at[row, pl.ds(0, BQ)], dA)
    pltpu.async_copy(tbl.at[sA.at[0]], rows0, sg0)

    @pl.loop(0, nblk // 2)
    def _(sb):
        for bb in range(2):
            b = 2 * sb + bb
            sI, dI = (sA, dA) if bb == 0 else (sB, dB)
            sO, dO = (sB, dB) if bb == 0 else (sA, dA)
            for p in range(BQ // 2):
                j0 = 2 * p
                j1 = 2 * p + 1

                if p == 0:
                    @pl.when(b > 0)
                    def _():
                        pltpu.make_async_copy(
                            rows1, acc.at[dO.at[BQ - 1]], ss1).wait()

                    @pl.when(b + 1 < nblk)
                    def _():
                        pltpu.sync_copy(
                            src_hbm.at[row, pl.ds((b + 1) * BQ, BQ)], sO)
                        pltpu.sync_copy(
                            dst_hbm.at[row, pl.ds((b + 1) * BQ, BQ)], dO)
                else:
                    pltpu.make_async_copy(
                        rows1, acc.at[dI.at[j1 - 2]], ss1).wait()

                pltpu.async_copy(tbl.at[sI.at[j1]], rows1, sg1)
                pltpu.make_async_copy(tbl.at[sI.at[j0]], rows0, sg0).wait()
                pltpu.async_copy(rows0, acc.at[dI.at[j0]], ss0, add=True)
                pltpu.make_async_copy(rows0, acc.at[dI.at[j0]], ss0).wait()

                if p < BQ // 2 - 1:
                    pltpu.async_copy(tbl.at[sI.at[j0 + 2]], rows0, sg0)
                else:
                    @pl.when(b + 1 < nblk)
                    def _():
                        pltpu.async_copy(tbl.at[sO.at[0]], rows0, sg0)

                pltpu.make_async_copy(tbl.at[sI.at[j1]], rows1, sg1).wait()
                pltpu.async_copy(rows1, acc.at[dI.at[j1]], ss1, add=True)

    pltpu.make_async_copy(rows1, acc.at[dB.at[BQ - 1]], ss1).wait()


@functools.partial(
    pl.kernel,
    out_type=jax.ShapeDtypeStruct((NC, NPA, D_IN), jnp.float32),
    mesh=_mesh(),
    scratch_types=[
        pltpu.VMEM((BQ, G2), jnp.int32),
        pltpu.VMEM((BQ, G2), jnp.int32),
        pltpu.VMEM((BQ, G2), jnp.int32),
        pltpu.VMEM((BQ, G2), jnp.int32),
        pltpu.VMEM((G2, D_IN), jnp.float32),
        pltpu.VMEM((G2, D_IN), jnp.float32),
        pltpu.VMEM_SHARED((NPA, D_IN), jnp.float32),
        pltpu.SemaphoreType.DMA,
        pltpu.SemaphoreType.DMA,
        pltpu.SemaphoreType.DMA,
        pltpu.SemaphoreType.DMA,
    ],
)
def _agg1_kernel(xs_hbm, src_hbm, dst_hbm, out_hbm,
                 sA, dA, sB, dB, rows0, rows1, acc, sg0, sg1, ss0, ss1):
    c = lax.axis_index("c")
    s = lax.axis_index("s")
    wid = s * NC + c

    _zero_acc(rows0, acc, s)
    plsc.subcore_barrier()
    _agg_blocks(xs_hbm, src_hbm, dst_hbm, wid, CT2 // BQ,
                sA, dA, sB, dB, rows0, rows1, acc, sg0, sg1, ss0, ss1)
    plsc.subcore_barrier()
    pltpu.sync_copy(acc.at[pl.ds(s * RPT, RPT)], out_hbm.at[c, pl.ds(s * RPT, RPT)])


@functools.partial(
    pl.kernel,
    out_type=jax.ShapeDtypeStruct((NC, NPA, D_IN), jnp.float32),
    mesh=_mesh(),
    scratch_types=[
        pltpu.VMEM((BQ, G2), jnp.int32),
        pltpu.VMEM((BQ, G2), jnp.int32),
        pltpu.VMEM((BQ, G2), jnp.int32),
        pltpu.VMEM((BQ, G2), jnp.int32),
        pltpu.VMEM((G2, D_IN), jnp.float32),
        pltpu.VMEM((G2, D_IN), jnp.float32),
        pltpu.VMEM_SHARED((NPA, D_IN), jnp.float32),
        pltpu.SemaphoreType.DMA,
        pltpu.SemaphoreType.DMA,
        pltpu.SemaphoreType.DMA,
        pltpu.SemaphoreType.DMA,
    ],
)
def _agg2_kernel(hs_hbm, src_hbm, dst_hbm, out_hbm,
                 sA, dA, sB, dB, rows0, rows1, acc, sg0, sg1, ss0, ss1):
    c = lax.axis_index("c")
    s = lax.axis_index("s")

    _zero_acc(rows0, acc, s)
    plsc.subcore_barrier()
    _agg_blocks(hs_hbm.at[c], src_hbm, dst_hbm, s, CS2 // BQ,
                sA, dA, sB, dB, rows0, rows1, acc, sg0, sg1, ss0, ss1)
    plsc.subcore_barrier()
    pltpu.sync_copy(acc.at[pl.ds(s * RPT, RPT)], out_hbm.at[c, pl.ds(s * RPT, RPT)])


PBR = G // 8
@functools.partial(
    pl.kernel,
    out_type=jax.ShapeDtypeStruct((NW, CPT, PBR, 128), jnp.float32),
    mesh=_mesh(),
    scratch_types=[
        pltpu.VMEM((CPT, G), jnp.int32),
        pltpu.VMEM((CPT, G), jnp.int32),
        pltpu.VMEM((G, N_GPS), jnp.float32),
        pltpu.VMEM((G, N_GPS), jnp.float32),
        pltpu.VMEM((G, N_GPS), jnp.float32),
        pltpu.VMEM((G, N_GPS), jnp.float32),
        pltpu.VMEM((G, N_GPS), jnp.float32),
        pltpu.VMEM((G, N_GPS), jnp.float32),
        pltpu.VMEM((PBR, 128), jnp.float32),
        pltpu.VMEM((PBR, 128), jnp.float32),
        pltpu.VMEM((PBR, 128), jnp.float32),
        pltpu.SemaphoreType.DMA,
        pltpu.SemaphoreType.DMA,
        pltpu.SemaphoreType.DMA,
        pltpu.SemaphoreType.DMA,
        pltpu.SemaphoreType.DMA,
        pltpu.SemaphoreType.DMA,
        pltpu.SemaphoreType.DMA,
        pltpu.SemaphoreType.DMA,
        pltpu.SemaphoreType.DMA,
    ],
)
def _logits_kernel(mu_hbm, src_hbm, dst_hbm, out_hbm, src_v, dst_v,
                   ra0, rb0, ra1, rb1, ra2, rb2, pb0, pb1, pb2,
                   sa0, sb0, sa1, sb1, sa2, sb2, so0, so1, so2):
    c = lax.axis_index("c")
    s = lax.axis_index("s")
    wid = s * NC + c

    ras = (ra0, ra1, ra2)
    rbs = (rb0, rb1, rb2)
    pbs = (pb0, pb1, pb2)
    sas = (sa0, sa1, sa2)
    sbs = (sb0, sb1, sb2)
    sos = (so0, so1, so2)

    pltpu.sync_copy(src_hbm.at[wid], src_v)
    pltpu.sync_copy(dst_hbm.at[wid], dst_v)

    def compute(ra, rb, pb):
        @pl.loop(0, PBR)
        def _(g8):
            for j in range(8):
                e = g8 * 8 + j
                acc = ra[e, pl.ds(0, L)] * rb[e, pl.ds(0, L)]
                for k in range(1, N_GPS // L):
                    acc = acc + ra[e, pl.ds(k * L, L)] * rb[e, pl.ds(k * L, L)]
                pb[g8, pl.ds(j * L, L)] = acc

    def handle(j, a, first, last):
        n = (a + 2) % 3
        if not last:
            pltpu.async_copy(mu_hbm.at[src_v.at[j + 2]], ras[n], sas[n])
            pltpu.async_copy(mu_hbm.at[dst_v.at[j + 2]], rbs[n], sbs[n])
        pltpu.make_async_copy(mu_hbm.at[src_v.at[j]], ras[a], sas[a]).wait()
        pltpu.make_async_copy(mu_hbm.at[dst_v.at[j]], rbs[a], sbs[a]).wait()
        if not first:
            pltpu.make_async_copy(pbs[a], out_hbm.at[wid, j - 3], sos[a]).wait()
        compute(ras[a], rbs[a], pbs[a])
        pltpu.async_copy(pbs[a], out_hbm.at[wid, j], sos[a])

    pltpu.async_copy(mu_hbm.at[src_v.at[0]], ra0, sa0)
    pltpu.async_copy(mu_hbm.at[dst_v.at[0]], rb0, sb0)
    pltpu.async_copy(mu_hbm.at[src_v.at[1]], ra1, sa1)
    pltpu.async_copy(mu_hbm.at[dst_v.at[1]], rb1, sb1)

    handle(0, 0, True, False)
    handle(1, 1, True, False)
    handle(2, 2, True, False)

    @pl.loop(0, CPT // 3 - 1)
    def _(t):
        j = 3 * t + 3
        handle(j, 0, False, False)
        handle(j + 1, 1, False, False)
        handle(j + 2, 2, False, False)

    handle(CPT - 2, 0, False, True)
    handle(CPT - 1, 1, False, True)

    pltpu.make_async_copy(pb2, out_hbm.at[wid, CPT - 3], so2).wait()
    pltpu.make_async_copy(pb0, out_hbm.at[wid, CPT - 2], so0).wait()
    pltpu.make_async_copy(pb1, out_hbm.at[wid, CPT - 1], so1).wait()


_BR = 1000


def _prep_tc(x, deg_a, deg_b):
    def body(x_ref, da_ref, db_ref, xs_ref, dv_ref):
        dinv = lax.rsqrt(da_ref[...] + db_ref[...] + 1.0)
        dv_ref[...] = dinv
        xs_ref[...] = jnp.log1p(x_ref[...]) * dinv

    return pl.pallas_call(
        body,
        grid=(N // _BR,),
        in_specs=[
            pl.BlockSpec((_BR, D_IN), lambda i: (i, 0)),
            pl.BlockSpec((_BR, 1), lambda i: (i, 0)),
            pl.BlockSpec((_BR, 1), lambda i: (i, 0)),
        ],
        out_specs=[
            pl.BlockSpec((_BR, D_IN), lambda i: (i, 0)),
            pl.BlockSpec((_BR, 1), lambda i: (i, 0)),
        ],
        out_shape=[
            jax.ShapeDtypeStruct((N, D_IN), jnp.float32),
            jax.ShapeDtypeStruct((N, 1), jnp.float32),
        ],
    )(x, deg_a, deg_b)


def _hidden_tc(raw1, xs, dinv, W1, b1):
    def body(r_ref, xs_ref, dv_ref, w_ref, b_ref, hs_ref):
        dinv = dv_ref[...]
        agg1 = dinv * (r_ref[0] + r_ref[1] + xs_ref[...])
        h = jnp.dot(agg1, w_ref[...], preferred_element_type=jnp.float32)
        h = jnp.maximum(h + b_ref[...], 0.0)
        hs = h * dinv
        hs_ref[0] = hs[:, :D_IN]
        hs_ref[1] = hs[:, D_IN:]

    return pl.pallas_call(
        body,
        grid=(N // _BR,),
        in_specs=[
            pl.BlockSpec((NC, _BR, D_IN), lambda i: (0, i, 0)),
            pl.BlockSpec((_BR, D_IN), lambda i: (i, 0)),
            pl.BlockSpec((_BR, 1), lambda i: (i, 0)),
            pl.BlockSpec((D_IN, D_H), lambda i: (0, 0)),
            pl.BlockSpec((1, D_H), lambda i: (0, 0)),
        ],
        out_specs=[pl.BlockSpec((NC, _BR, D_IN), lambda i: (0, i, 0))],
        out_shape=[jax.ShapeDtypeStruct((NC, N, D_IN), jnp.float32)],
    )(raw1, xs, dinv, W1, b1)[0]


def _out_tc(raw2, hs2, dinv, Wmu, Wls, Wdec, mask):
    def body(r_ref, hs_ref, dv_ref, wmu_ref, wls_ref, wd_ref, m_ref,
             mu_ref, ls_ref, nb_ref):
        dinv = dv_ref[...]
        a_lo = dinv * (r_ref[0] + hs_ref[0])
        a_hi = dinv * (r_ref[1] + hs_ref[1])
        wmu = wmu_ref[...]
        wls = wls_ref[...]
        mu = jnp.dot(a_lo, wmu[:D_IN], preferred_element_type=jnp.float32)
        mu = mu + jnp.dot(a_hi, wmu[D_IN:], preferred_element_type=jnp.float32)
        ls = jnp.dot(a_lo, wls[:D_IN], preferred_element_type=jnp.float32)
        ls = ls + jnp.dot(a_hi, wls[D_IN:], preferred_element_type=jnp.float32)
        mu_ref[...] = mu
        ls_ref[...] = ls
        wm = wd_ref[...] * m_ref[...]
        nb = jnp.dot(mu, wm, preferred_element_type=jnp.float32)
        nb_ref[...] = jnp.exp(jnp.clip(nb, -10.0, 10.0))

    return pl.pallas_call(
        body,
        grid=(N // _BR,),
        in_specs=[
            pl.BlockSpec((NC, _BR, D_IN), lambda i: (0, i, 0)),
            pl.BlockSpec((NC, _BR, D_IN), lambda i: (0, i, 0)),
            pl.BlockSpec((_BR, 1), lambda i: (i, 0)),
            pl.BlockSpec((D_H, N_GPS), lambda i: (0, 0)),
            pl.BlockSpec((D_H, N_GPS), lambda i: (0, 0)),
            pl.BlockSpec((N_GPS, D_IN), lambda i: (0, 0)),
            pl.BlockSpec((N_GPS, D_IN), lambda i: (0, 0)),
        ],
        out_specs=[
            pl.BlockSpec((_BR, N_GPS), lambda i: (i, 0)),
            pl.BlockSpec((_BR, N_GPS), lambda i: (i, 0)),
            pl.BlockSpec((_BR, D_IN), lambda i: (i, 0)),
        ],
        out_shape=[
            jax.ShapeDtypeStruct((N, N_GPS), jnp.float32),
            jax.ShapeDtypeStruct((N, N_GPS), jnp.float32),
            jax.ShapeDtypeStruct((N, D_IN), jnp.float32),
        ],
    )(raw2, hs2, dinv, Wmu, Wls, Wdec, mask)


def _lsum_tc(partial):
    _BE = 4000

    def body(p_ref, o_ref):
        d = lax.broadcasted_iota(jnp.int32, (128, 8), 0)
        j = lax.broadcasted_iota(jnp.int32, (128, 8), 1)
        sel = (d // L == j).astype(jnp.float32)
        o_ref[...] = jnp.dot(p_ref[...], sel, preferred_element_type=jnp.float32)

    return pl.pallas_call(
        body,
        grid=(E // 8 // _BE,),
        in_specs=[pl.BlockSpec((_BE, 128), lambda i: (i, 0))],
        out_specs=[pl.BlockSpec((_BE, 8), lambda i: (i, 0))],
        out_shape=[jax.ShapeDtypeStruct((E // 8, 8), jnp.float32)],
    )(partial)[0]


def kernel(x, edge_index, W1, b1, Wmu, Wls, Wdec, mask):
    src = edge_index[0].astype(jnp.int32)
    dst = edge_index[1].astype(jnp.int32)
    src32 = src.reshape(NW, CPT, G)
    dst32 = dst.reshape(NW, CPT, G)
    src32a = src.reshape(NW, CT2, G2)
    dst32a = dst.reshape(NW, CT2, G2)
    src16a = src.reshape(NS, CS2, G2)
    dst16a = dst.reshape(NS, CS2, G2)
    b1r = b1.reshape(1, D_H)

    deg2 = _deg_kernel(dst32)
    deg_a = deg2[0, 0, :N].reshape(N, 1)
    deg_b = deg2[1, 0, :N].reshape(N, 1)
    xs, dinv = _prep_tc(x, deg_a, deg_b)
    raw1 = _agg1_kernel(xs, src32a, dst32a)
    hs2 = _hidden_tc(raw1, xs, dinv, W1, b1r)
    raw2 = _agg2_kernel(hs2, src16a, dst16a)
    mu, logstd, nb_mean = _out_tc(raw2, hs2, dinv, Wmu, Wls, Wdec, mask)
    lpart = _logits_kernel(mu, src32, dst32).reshape(E // 8, 128)
    edge_logits = _lsum_tc(lpart).reshape(E)
    return mu, logstd, edge_logits, nb_mean
